# baseline (speedup 1.0000x reference)
import functools

import numpy as np
import jax
import jax.numpy as jnp
from jax.experimental import pallas as pl
from jax.experimental.pallas import tpu as pltpu

_SUB = 8


def _sample_row_kernel(idx_ref, key_ref, p_ref, a_ref, lp_ref, sp_ref, *, vocab, lanes):
    b = pl.program_id(0)
    p = p_ref[0]
    row = jax.lax.broadcasted_iota(jnp.int32, (_SUB, lanes), 0)
    col = jax.lax.broadcasted_iota(jnp.int32, (_SUB, lanes), 1)
    v = row * jnp.int32(lanes) + col
    i = (b * jnp.int32(vocab) + v).astype(jnp.uint32)

    k0 = key_ref[0]
    k1 = key_ref[1]
    ks2 = k0 ^ k1 ^ jnp.uint32(0x1BD11BDA)
    ks = (k0, k1, ks2)
    x0 = jnp.full((_SUB, lanes), k0, jnp.uint32)
    x1 = i + k1
    rotations = ((13, 15, 26, 6), (17, 29, 16, 24))
    for g in range(5):
        for r in rotations[g % 2]:
            x0 = x0 + x1
            x1 = (x1 << jnp.uint32(r)) | (x1 >> jnp.uint32(32 - r))
            x1 = x1 ^ x0
        x0 = x0 + ks[(g + 1) % 3]
        x1 = x1 + ks[(g + 2) % 3] + jnp.uint32(g + 1)
    bits = x0 ^ x1

    f = jax.lax.bitcast_convert_type(
        (bits >> jnp.uint32(9)) | jnp.uint32(0x3F800000), jnp.float32
    ) - jnp.float32(1.0)
    tiny = jnp.float32(np.finfo(np.float32).tiny)
    t = -jnp.log(jnp.maximum(f, tiny))

    score = p / t
    total = jnp.sum(p)
    m = jnp.max(score)
    action = jnp.min(jnp.where(score == m, v, jnp.int32(0x7FFFFFFF)))
    sel = jnp.sum(jnp.where(v == action, p, jnp.float32(0.0)))

    a_ref[0, 0, 0] = action
    lp_ref[0, 0, 0] = jnp.log(sel) - jnp.log(total)
    sp_ref[0, 0, 0] = sel


def kernel(probs_table, indices):
    num_states, vocab = probs_table.shape
    batch = indices.shape[0]
    assert vocab % _SUB == 0
    lanes = vocab // _SUB
    table3 = probs_table.reshape(num_states, _SUB, lanes)

    key_data = jax.random.key_data(
        jax.random.fold_in(jax.random.key(0), 123)
    ).astype(jnp.uint32)

    grid_spec = pltpu.PrefetchScalarGridSpec(
        num_scalar_prefetch=2,
        grid=(batch,),
        in_specs=[
            pl.BlockSpec((1, _SUB, lanes), lambda b, idx, key: (idx[b], 0, 0)),
        ],
        out_specs=[
            pl.BlockSpec((1, 1, 1), lambda b, idx, key: (b, 0, 0), memory_space=pltpu.SMEM),
            pl.BlockSpec((1, 1, 1), lambda b, idx, key: (b, 0, 0), memory_space=pltpu.SMEM),
            pl.BlockSpec((1, 1, 1), lambda b, idx, key: (b, 0, 0), memory_space=pltpu.SMEM),
        ],
    )
    actions, log_probs, selected = pl.pallas_call(
        functools.partial(_sample_row_kernel, vocab=vocab, lanes=lanes),
        grid_spec=grid_spec,
        out_shape=[
            jax.ShapeDtypeStruct((batch, 1, 1), jnp.int32),
            jax.ShapeDtypeStruct((batch, 1, 1), jnp.float32),
            jax.ShapeDtypeStruct((batch, 1, 1), jnp.float32),
        ],
    )(indices, key_data, table3)
    return actions[:, 0, 0], log_probs[:, 0, 0], selected[:, 0, 0]


# speedup vs baseline: 1.1888x; 1.1888x over previous
import functools

import numpy as np
import jax
import jax.numpy as jnp
from jax.experimental import pallas as pl
from jax.experimental.pallas import tpu as pltpu

_SUB = 8
_W = 512

_INT_MAX = np.int32(0x7FFFFFFF)


def _threefry_score(p, v, base_u, k0, k1, ks2, width):
    i = base_u + v.astype(jnp.uint32)
    ks = (k0, k1, ks2)
    x0 = jnp.broadcast_to(k0, (_SUB, width))
    x1 = i + k1
    rotations = ((13, 15, 26, 6), (17, 29, 16, 24))
    for g in range(5):
        for r in rotations[g % 2]:
            x0 = x0 + x1
            x1 = (x1 << jnp.uint32(r)) | (x1 >> jnp.uint32(32 - r))
            x1 = x1 ^ x0
        x0 = x0 + ks[(g + 1) % 3]
        x1 = x1 + (ks[(g + 2) % 3] + jnp.uint32(g + 1))
    bits = x0 ^ x1
    f = jax.lax.bitcast_convert_type(
        (bits >> jnp.uint32(9)) | jnp.uint32(0x3F800000), jnp.float32
    ) - jnp.float32(1.0)
    tiny = jnp.float32(np.finfo(np.float32).tiny)
    t = -jnp.log(jnp.maximum(f, tiny))
    return p / t


def _sample_row_kernel(idx_ref, key_ref, p_ref, a_ref, lp_ref, sp_ref, *, vocab, lanes):
    b = pl.program_id(0)
    base_u = (b * jnp.int32(vocab)).astype(jnp.uint32)
    k0 = key_ref[0]
    k1 = key_ref[1]
    ks2 = k0 ^ k1 ^ jnp.uint32(0x1BD11BDA)

    nfull = lanes // _W
    tail = lanes - nfull * _W

    rowi = jax.lax.broadcasted_iota(jnp.int32, (_SUB, _W), 0)
    lane = jax.lax.broadcasted_iota(jnp.int32, (_SUB, _W), 1)
    v0 = rowi * jnp.int32(lanes) + lane

    m_acc = jnp.full((_SUB, _W), -jnp.inf, jnp.float32)
    a_acc = jnp.zeros((_SUB, _W), jnp.int32)
    ps_acc = jnp.zeros((_SUB, _W), jnp.float32)
    s_acc = jnp.zeros((_SUB, _W), jnp.float32)

    for kc in range(nfull):
        v = v0 + jnp.int32(kc * _W)
        p = p_ref[0, :, kc * _W:(kc + 1) * _W]
        s = _threefry_score(p, v, base_u, k0, k1, ks2, _W)
        upd = s > m_acc
        m_acc = jnp.where(upd, s, m_acc)
        a_acc = jnp.where(upd, v, a_acc)
        ps_acc = jnp.where(upd, p, ps_acc)
        s_acc = s_acc + p

    m1 = jnp.max(m_acc)
    win = m_acc == m1
    a1 = jnp.min(jnp.where(win, a_acc, _INT_MAX))
    sel1 = jnp.sum(jnp.where(win & (a_acc == a1), ps_acc, jnp.float32(0.0)))
    total = jnp.sum(s_acc)

    if tail:
        vt = (v0 + jnp.int32(nfull * _W))[:, :tail]
        pt = p_ref[0, :, nfull * _W:lanes]
        st = _threefry_score(pt, vt, base_u, k0, k1, ks2, tail)
        m2 = jnp.max(st)
        a2 = jnp.min(jnp.where(st == m2, vt, _INT_MAX))
        sel2 = jnp.sum(jnp.where(vt == a2, pt, jnp.float32(0.0)))
        total = total + jnp.sum(pt)
        better2 = (m2 > m1) | ((m2 == m1) & (a2 < a1))
        a1 = jnp.where(better2, a2, a1)
        sel1 = jnp.where(better2, sel2, sel1)

    a_ref[0, 0, 0] = a1
    lp_ref[0, 0, 0] = jnp.log(sel1) - jnp.log(total)
    sp_ref[0, 0, 0] = sel1


def kernel(probs_table, indices):
    num_states, vocab = probs_table.shape
    batch = indices.shape[0]
    assert vocab % _SUB == 0
    lanes = vocab // _SUB
    table3 = probs_table.reshape(num_states, _SUB, lanes)

    key_data = jax.random.key_data(
        jax.random.fold_in(jax.random.key(0), 123)
    ).astype(jnp.uint32)

    grid_spec = pltpu.PrefetchScalarGridSpec(
        num_scalar_prefetch=2,
        grid=(batch,),
        in_specs=[
            pl.BlockSpec((1, _SUB, lanes), lambda b, idx, key: (idx[b], 0, 0)),
        ],
        out_specs=[
            pl.BlockSpec((1, 1, 1), lambda b, idx, key: (b, 0, 0), memory_space=pltpu.SMEM),
            pl.BlockSpec((1, 1, 1), lambda b, idx, key: (b, 0, 0), memory_space=pltpu.SMEM),
            pl.BlockSpec((1, 1, 1), lambda b, idx, key: (b, 0, 0), memory_space=pltpu.SMEM),
        ],
    )
    actions, log_probs, selected = pl.pallas_call(
        functools.partial(_sample_row_kernel, vocab=vocab, lanes=lanes),
        grid_spec=grid_spec,
        out_shape=[
            jax.ShapeDtypeStruct((batch, 1, 1), jnp.int32),
            jax.ShapeDtypeStruct((batch, 1, 1), jnp.float32),
            jax.ShapeDtypeStruct((batch, 1, 1), jnp.float32),
        ],
    )(indices, key_data, table3)
    return actions[:, 0, 0], log_probs[:, 0, 0], selected[:, 0, 0]


# speedup vs baseline: 1.2248x; 1.0303x over previous
import functools

import numpy as np
import jax
import jax.numpy as jnp
from jax.experimental import pallas as pl
from jax.experimental.pallas import tpu as pltpu

_SUB = 8
_W = 512

_INT_MAX = np.int32(0x7FFFFFFF)


def _threefry_score(p, v, base_u, k0, k1, ks2, width):
    i = base_u + v.astype(jnp.uint32)
    ks = (k0, k1, ks2)
    x0 = jnp.broadcast_to(k0, (_SUB, width))
    x1 = i + k1
    rotations = ((13, 15, 26, 6), (17, 29, 16, 24))
    for g in range(5):
        for r in rotations[g % 2]:
            x0 = x0 + x1
            x1 = (x1 << jnp.uint32(r)) | (x1 >> jnp.uint32(32 - r))
            x1 = x1 ^ x0
        x0 = x0 + ks[(g + 1) % 3]
        x1 = x1 + (ks[(g + 2) % 3] + jnp.uint32(g + 1))
    bits = x0 ^ x1
    f = jax.lax.bitcast_convert_type(
        (bits >> jnp.uint32(9)) | jnp.uint32(0x3F800000), jnp.float32
    ) - jnp.float32(1.0)
    tiny = jnp.float32(np.finfo(np.float32).tiny)
    t = -jnp.log(jnp.maximum(f, tiny))
    return p / t


def _sample_rows_kernel(idx_ref, key_ref, *refs, vocab, lanes, rows):
    p_refs = refs[:rows]
    a_ref, lp_ref, sp_ref = refs[rows:]
    b = pl.program_id(0)
    k0 = key_ref[0]
    k1 = key_ref[1]
    ks2 = k0 ^ k1 ^ jnp.uint32(0x1BD11BDA)

    nfull = lanes // _W
    tail = lanes - nfull * _W

    rowi = jax.lax.broadcasted_iota(jnp.int32, (_SUB, _W), 0)
    lane = jax.lax.broadcasted_iota(jnp.int32, (_SUB, _W), 1)
    v0 = rowi * jnp.int32(lanes) + lane

    for j, p_ref in enumerate(p_refs):
        base_u = ((b * jnp.int32(rows) + jnp.int32(j)) * jnp.int32(vocab)).astype(jnp.uint32)
        m_acc = jnp.full((_SUB, _W), -jnp.inf, jnp.float32)
        a_acc = jnp.zeros((_SUB, _W), jnp.int32)
        ps_acc = jnp.zeros((_SUB, _W), jnp.float32)
        s_acc = jnp.zeros((_SUB, _W), jnp.float32)

        for kc in range(nfull):
            v = v0 + jnp.int32(kc * _W)
            p = p_ref[0, :, kc * _W:(kc + 1) * _W]
            s = _threefry_score(p, v, base_u, k0, k1, ks2, _W)
            upd = s > m_acc
            m_acc = jnp.where(upd, s, m_acc)
            a_acc = jnp.where(upd, v, a_acc)
            ps_acc = jnp.where(upd, p, ps_acc)
            s_acc = s_acc + p

        m1 = jnp.max(m_acc)
        win = m_acc == m1
        a1 = jnp.min(jnp.where(win, a_acc, _INT_MAX))
        sel1 = jnp.sum(jnp.where(win & (a_acc == a1), ps_acc, jnp.float32(0.0)))
        total = jnp.sum(s_acc)

        if tail:
            vt = (v0 + jnp.int32(nfull * _W))[:, :tail]
            pt = p_ref[0, :, nfull * _W:lanes]
            st = _threefry_score(pt, vt, base_u, k0, k1, ks2, tail)
            m2 = jnp.max(st)
            a2 = jnp.min(jnp.where(st == m2, vt, _INT_MAX))
            sel2 = jnp.sum(jnp.where(vt == a2, pt, jnp.float32(0.0)))
            total = total + jnp.sum(pt)
            better2 = (m2 > m1) | ((m2 == m1) & (a2 < a1))
            a1 = jnp.where(better2, a2, a1)
            sel1 = jnp.where(better2, sel2, sel1)

        a_ref[j, 0, 0] = a1
        lp_ref[j, 0, 0] = jnp.log(sel1) - jnp.log(total)
        sp_ref[j, 0, 0] = sel1


_ROWS = 2


def kernel(probs_table, indices):
    num_states, vocab = probs_table.shape
    batch = indices.shape[0]
    assert vocab % _SUB == 0 and batch % _ROWS == 0
    lanes = vocab // _SUB
    table3 = probs_table.reshape(num_states, _SUB, lanes)

    key_data = jax.random.key_data(
        jax.random.fold_in(jax.random.key(0), 123)
    ).astype(jnp.uint32)

    def _in_spec(j):
        return pl.BlockSpec(
            (1, _SUB, lanes), lambda b, idx, key, j=j: (idx[b * _ROWS + j], 0, 0)
        )

    grid_spec = pltpu.PrefetchScalarGridSpec(
        num_scalar_prefetch=2,
        grid=(batch // _ROWS,),
        in_specs=[_in_spec(j) for j in range(_ROWS)],
        out_specs=[
            pl.BlockSpec((_ROWS, 1, 1), lambda b, idx, key: (b, 0, 0), memory_space=pltpu.SMEM),
            pl.BlockSpec((_ROWS, 1, 1), lambda b, idx, key: (b, 0, 0), memory_space=pltpu.SMEM),
            pl.BlockSpec((_ROWS, 1, 1), lambda b, idx, key: (b, 0, 0), memory_space=pltpu.SMEM),
        ],
    )
    actions, log_probs, selected = pl.pallas_call(
        functools.partial(_sample_rows_kernel, vocab=vocab, lanes=lanes, rows=_ROWS),
        grid_spec=grid_spec,
        out_shape=[
            jax.ShapeDtypeStruct((batch, 1, 1), jnp.int32),
            jax.ShapeDtypeStruct((batch, 1, 1), jnp.float32),
            jax.ShapeDtypeStruct((batch, 1, 1), jnp.float32),
        ],
    )(indices, key_data, *([table3] * _ROWS))
    return actions[:, 0, 0], log_probs[:, 0, 0], selected[:, 0, 0]


# speedup vs baseline: 1.5909x; 1.2989x over previous
import functools

import numpy as np
import jax
import jax.numpy as jnp
from jax.experimental import pallas as pl
from jax.experimental.pallas import tpu as pltpu

_SUB = 8
_W = 512
_CH = _SUB * _W

_INT_MAX = np.int32(0x7FFFFFFF)


def _threefry_score(p, v, base_u, k0, k1, ks2, width):
    i = base_u + v.astype(jnp.uint32)
    ks = (k0, k1, ks2)
    x0 = jnp.broadcast_to(k0, (_SUB, width))
    x1 = i + k1
    rotations = ((13, 15, 26, 6), (17, 29, 16, 24))
    for g in range(5):
        for r in rotations[g % 2]:
            x0 = x0 + x1
            x1 = (x1 << jnp.uint32(r)) | (x1 >> jnp.uint32(32 - r))
            x1 = x1 ^ x0
        x0 = x0 + ks[(g + 1) % 3]
        x1 = x1 + (ks[(g + 2) % 3] + jnp.uint32(g + 1))
    bits = x0 ^ x1
    f = jax.lax.bitcast_convert_type(
        (bits >> jnp.uint32(9)) | jnp.uint32(0x3F800000), jnp.float32
    ) - jnp.float32(1.0)
    tiny = jnp.float32(np.finfo(np.float32).tiny)
    t = -jnp.log(jnp.maximum(f, tiny))
    return p / t


def _sample_kernel(idx_ref, key_ref, p_ref, a_ref, lp_ref, sp_ref, *, vocab):
    b = pl.program_id(0)
    row = idx_ref[b]
    s = row - (row // _SUB) * _SUB

    k0 = key_ref[0]
    k1 = key_ref[1]
    ks2 = k0 ^ k1 ^ jnp.uint32(0x1BD11BDA)
    base_u = (b * jnp.int32(vocab)).astype(jnp.uint32)

    nfull = vocab // _CH
    tailn = vocab - nfull * _CH
    tail_w = tailn // _SUB

    rowi = jax.lax.broadcasted_iota(jnp.int32, (_SUB, _W), 0)
    lane = jax.lax.broadcasted_iota(jnp.int32, (_SUB, _W), 1)
    v0 = rowi * jnp.int32(_W) + lane

    m_acc = jnp.full((_SUB, _W), -jnp.inf, jnp.float32)
    a_acc = jnp.zeros((_SUB, _W), jnp.int32)
    ps_acc = jnp.zeros((_SUB, _W), jnp.float32)
    s_acc = jnp.zeros((_SUB, _W), jnp.float32)

    def extract(c0, width):
        pieces = [
            p_ref[0, pl.ds(s, 1), c0 + r * width:c0 + (r + 1) * width]
            for r in range(_SUB)
        ]
        return jnp.concatenate(pieces, axis=0)

    for kc in range(nfull):
        v = v0 + jnp.int32(kc * _CH)
        p = extract(kc * _CH, _W)
        sc = _threefry_score(p, v, base_u, k0, k1, ks2, _W)
        upd = sc > m_acc
        m_acc = jnp.where(upd, sc, m_acc)
        a_acc = jnp.where(upd, v, a_acc)
        ps_acc = jnp.where(upd, p, ps_acc)
        s_acc = s_acc + p

    m1 = jnp.max(m_acc)
    win = m_acc == m1
    a1 = jnp.min(jnp.where(win, a_acc, _INT_MAX))
    sel1 = jnp.sum(jnp.where(win & (a_acc == a1), ps_acc, jnp.float32(0.0)))
    total = jnp.sum(s_acc)

    if tailn:
        rowt = jax.lax.broadcasted_iota(jnp.int32, (_SUB, tail_w), 0)
        lanet = jax.lax.broadcasted_iota(jnp.int32, (_SUB, tail_w), 1)
        vt = rowt * jnp.int32(tail_w) + lanet + jnp.int32(nfull * _CH)
        pt = extract(nfull * _CH, tail_w)
        st = _threefry_score(pt, vt, base_u, k0, k1, ks2, tail_w)
        m2 = jnp.max(st)
        a2 = jnp.min(jnp.where(st == m2, vt, _INT_MAX))
        sel2 = jnp.sum(jnp.where(vt == a2, pt, jnp.float32(0.0)))
        total = total + jnp.sum(pt)
        better2 = (m2 > m1) | ((m2 == m1) & (a2 < a1))
        a1 = jnp.where(better2, a2, a1)
        sel1 = jnp.where(better2, sel2, sel1)

    a_ref[0, 0, 0] = a1
    lp_ref[0, 0, 0] = jnp.log(sel1) - jnp.log(total)
    sp_ref[0, 0, 0] = sel1


def kernel(probs_table, indices):
    num_states, vocab = probs_table.shape
    batch = indices.shape[0]
    assert num_states % _SUB == 0 and vocab % _SUB == 0
    table_g = probs_table.reshape(num_states // _SUB, _SUB, vocab)

    key_data = jax.random.key_data(
        jax.random.fold_in(jax.random.key(0), 123)
    ).astype(jnp.uint32)

    grid_spec = pltpu.PrefetchScalarGridSpec(
        num_scalar_prefetch=2,
        grid=(batch,),
        in_specs=[
            pl.BlockSpec((1, _SUB, vocab), lambda b, idx, key: (idx[b] // _SUB, 0, 0)),
        ],
        out_specs=[
            pl.BlockSpec((1, 1, 1), lambda b, idx, key: (b, 0, 0), memory_space=pltpu.SMEM),
            pl.BlockSpec((1, 1, 1), lambda b, idx, key: (b, 0, 0), memory_space=pltpu.SMEM),
            pl.BlockSpec((1, 1, 1), lambda b, idx, key: (b, 0, 0), memory_space=pltpu.SMEM),
        ],
    )
    actions, log_probs, selected = pl.pallas_call(
        functools.partial(_sample_kernel, vocab=vocab),
        grid_spec=grid_spec,
        out_shape=[
            jax.ShapeDtypeStruct((batch, 1, 1), jnp.int32),
            jax.ShapeDtypeStruct((batch, 1, 1), jnp.float32),
            jax.ShapeDtypeStruct((batch, 1, 1), jnp.float32),
        ],
    )(indices, key_data, table_g)
    return actions[:, 0, 0], log_probs[:, 0, 0], selected[:, 0, 0]
